# Initial kernel scaffold; baseline (speedup 1.0000x reference)
import jax
import jax.numpy as jnp
from jax.experimental import pallas as pl


def kernel(x, Wr, W1, W2):
    raise NotImplementedError("write your pallas kernel here")


# speedup vs baseline: 3.4735x; 3.4735x over previous
import jax
import jax.numpy as jnp
from jax.experimental import pallas as pl
from jax.experimental.pallas import tpu as pltpu

_K = 2
_HC = 512


def _router_body(x_ref, wr_ref, gate_ref, lb_ref):
    t, e_num = gate_ref.shape
    x = x_ref[...]
    wr = wr_ref[...]
    logits = jax.lax.dot_general(
        x, wr, (((1,), (1,)), ((), ())),
        preferred_element_type=jnp.float32)
    m = jnp.max(logits, axis=1, keepdims=True)
    ex = jnp.exp(logits - m)
    probs = ex / jnp.sum(ex, axis=1, keepdims=True)
    lane = jax.lax.broadcasted_iota(jnp.int32, (t, e_num), 1)
    p1 = jnp.max(probs, axis=1, keepdims=True)
    i1 = jnp.min(jnp.where(probs == p1, lane, e_num), axis=1, keepdims=True)
    probs2 = jnp.where(lane == i1, -jnp.inf, probs)
    p2 = jnp.max(probs2, axis=1, keepdims=True)
    i2 = jnp.min(jnp.where(probs2 == p2, lane, e_num), axis=1, keepdims=True)
    s = p1 + p2 + 1e-9
    gate = jnp.where(lane == i1, p1 / s, jnp.where(lane == i2, p2 / s, 0.0))
    gate_ref[...] = gate
    sel = ((lane == i1) | (lane == i2)).astype(jnp.float32)
    imp = jnp.sum(probs, axis=0, keepdims=True)
    load = jnp.sum(sel, axis=0, keepdims=True)
    imp = imp / (jnp.sum(imp) + 1e-9)
    load = load / (jnp.sum(load) + 1e-9)
    lb_ref[0, 0] = jnp.sum(imp * load) * e_num


def _ffn_body(x_ref, w1_ref, w2_ref, gate_ref, out_ref):
    e = pl.program_id(0)
    c = pl.program_id(1)

    @pl.when((e == 0) & (c == 0))
    def _():
        out_ref[...] = jnp.zeros_like(out_ref)

    t = x_ref.shape[0]
    x = x_ref[...]
    w1 = w1_ref[0]
    w2 = w2_ref[0]
    h = jax.lax.dot_general(
        x, w1, (((1,), (1,)), ((), ())),
        preferred_element_type=jnp.float32)
    h = 0.5 * h * (1.0 + jax.lax.erf(h * 0.7071067811865476))
    y = jax.lax.dot_general(
        h, w2, (((1,), (1,)), ((), ())),
        preferred_element_type=jnp.float32)
    e_num = gate_ref.shape[1]
    onehot = (jax.lax.broadcasted_iota(jnp.int32, (e_num, 1), 0) == e
              ).astype(jnp.float32)
    g = jax.lax.dot_general(
        gate_ref[...], onehot, (((1,), (0,)), ((), ())),
        preferred_element_type=jnp.float32)
    out_ref[...] += y * g


def kernel(x, Wr, W1, W2):
    b, t, d = x.shape
    e_num, h_dim, _ = W1.shape
    nc = h_dim // _HC
    x2 = x.reshape(b * t, d)

    gate, lb = pl.pallas_call(
        _router_body,
        out_shape=[
            jax.ShapeDtypeStruct((b * t, e_num), jnp.float32),
            jax.ShapeDtypeStruct((1, 1), jnp.float32),
        ],
        out_specs=[
            pl.BlockSpec(memory_space=pltpu.VMEM),
            pl.BlockSpec(memory_space=pltpu.SMEM),
        ],
    )(x2, Wr)

    out = pl.pallas_call(
        _ffn_body,
        grid=(e_num, nc),
        in_specs=[
            pl.BlockSpec((b * t, d), lambda e, c: (0, 0)),
            pl.BlockSpec((1, _HC, d), lambda e, c: (e, c, 0)),
            pl.BlockSpec((1, d, _HC), lambda e, c: (e, 0, c)),
            pl.BlockSpec((b * t, e_num), lambda e, c: (0, 0)),
        ],
        out_specs=pl.BlockSpec((b * t, d), lambda e, c: (0, 0)),
        out_shape=jax.ShapeDtypeStruct((b * t, d), jnp.float32),
    )(x2, W1, W2, gate)

    return out.reshape(b, t, d), lb[0, 0]
